# baseline (speedup 1.0000x reference)
import jax
import jax.numpy as jnp
from jax.experimental import pallas as pl
from jax.experimental.pallas import tpu as pltpu

_THETA_ALPHA = 80.0
_THETA_BETA = 13.0
_THETA_GAMMA = 3.0
_BILATERAL_COMPAT = 10.0
_SPATIAL_COMPAT = 3.0
_NUM_ITERATIONS = 5
_BR = 128
_CPAD = 128
_NEG_BIG = 1.0e30


def _softmax_rows(x):
    m = jnp.max(x, axis=1, keepdims=True)
    e = jnp.exp(x - m)
    return e / jnp.sum(e, axis=1, keepdims=True)


def _crf_kernel(bi16_ref, sp16_ref, nsqb_col_ref, nsqs_col_ref,
                u_ref, out_ref, m_ref, c_ref, qa_ref, qb_ref):
    p = pl.program_id(0)
    n = m_ref.shape[1]
    nblocks = n // _BR

    @pl.when(p < nblocks)
    def _build():
        row0 = p * _BR

        def gauss(f16_ref, nsq_col_ref):
            cross = jax.lax.dot_general(
                f16_ref[pl.ds(row0, _BR), :], f16_ref[...],
                (((1,), (1,)), ((), ())),
                preferred_element_type=jnp.float32)
            nsq_row = jnp.reshape(nsq_col_ref[:, pl.ds(row0, _BR)], (_BR, 1))
            e = jnp.exp(jnp.minimum(nsq_row + nsq_col_ref[...] + cross, 0.0))
            norm = jnp.maximum(
                jnp.sum(e, axis=1, keepdims=True) - 1.0, 1e-20)
            return e, norm

        eb, nb = gauss(bi16_ref, nsqb_col_ref)
        es, ns = gauss(sp16_ref, nsqs_col_ref)
        sb = _BILATERAL_COMPAT / nb
        ss = _SPATIAL_COMPAT / ns
        m_ref[pl.ds(row0, _BR), :] = (sb * eb + ss * es).astype(jnp.bfloat16)
        c_ref[pl.ds(row0, _BR), :] = (sb + ss).astype(jnp.bfloat16)

        @pl.when(p == 0)
        def _init():
            qa_ref[...] = _softmax_rows(-u_ref[...]).astype(jnp.bfloat16)

    @pl.when(p >= nblocks)
    def _iterate():
        it = p - nblocks

        def step(src_ref, dst_ref):
            q16 = src_ref[...]
            wm = jax.lax.dot_general(
                m_ref[...], q16, (((1,), (0,)), ((), ())),
                preferred_element_type=jnp.float32)
            weighted = wm - c_ref[...].astype(jnp.float32) * q16.astype(jnp.float32)
            q_new = _softmax_rows(weighted - u_ref[...])
            dst_ref[...] = q_new.astype(jnp.bfloat16)

            @pl.when(it == _NUM_ITERATIONS - 1)
            def _emit():
                out_ref[...] = q_new

        @pl.when(it % 2 == 0)
        def _even():
            step(qa_ref, qb_ref)

        @pl.when(it % 2 == 1)
        def _odd():
            step(qb_ref, qa_ref)


def kernel(unary, image):
    hh, ww, cc = unary.shape
    n = hh * ww
    nblocks = n // _BR

    ys, xs = jnp.meshgrid(jnp.arange(hh, dtype=jnp.float32),
                          jnp.arange(ww, dtype=jnp.float32), indexing='ij')
    xs = xs.reshape(-1)
    ys = ys.reshape(-1)
    rgb = image.reshape(n, 3) * 255.0
    bi = jnp.concatenate([(xs / _THETA_ALPHA)[:, None],
                          (ys / _THETA_ALPHA)[:, None],
                          rgb / _THETA_BETA], axis=1)
    sp = jnp.stack([xs / _THETA_GAMMA, ys / _THETA_GAMMA], axis=1)
    bi = jnp.pad(bi, ((0, 0), (0, 3)))
    sp = jnp.pad(sp, ((0, 0), (0, 6)))
    nsqb = -0.5 * jnp.sum(bi * bi, axis=1)
    nsqs = -0.5 * jnp.sum(sp * sp, axis=1)

    u = unary.reshape(n, cc)
    u_pad = jnp.full((n, _CPAD), _NEG_BIG, dtype=jnp.float32)
    u_pad = u_pad.at[:, :cc].set(u)

    q = pl.pallas_call(
        _crf_kernel,
        grid=(nblocks + _NUM_ITERATIONS,),
        in_specs=[
            pl.BlockSpec((n, 8), lambda p: (0, 0)),
            pl.BlockSpec((n, 8), lambda p: (0, 0)),
            pl.BlockSpec((1, n), lambda p: (0, 0)),
            pl.BlockSpec((1, n), lambda p: (0, 0)),
            pl.BlockSpec((n, _CPAD), lambda p: (0, 0)),
        ],
        out_specs=pl.BlockSpec((n, _CPAD), lambda p: (0, 0)),
        out_shape=jax.ShapeDtypeStruct((n, _CPAD), jnp.float32),
        scratch_shapes=[
            pltpu.VMEM((n, n), jnp.bfloat16),
            pltpu.VMEM((n, 1), jnp.bfloat16),
            pltpu.VMEM((n, _CPAD), jnp.bfloat16),
            pltpu.VMEM((n, _CPAD), jnp.bfloat16),
        ],
        compiler_params=pltpu.CompilerParams(
            dimension_semantics=("arbitrary",)),
    )(bi.astype(jnp.bfloat16), sp.astype(jnp.bfloat16),
      nsqb.reshape(1, n), nsqs.reshape(1, n), u_pad)

    return q[:, :cc].reshape(hh, ww, cc)
